# Initial kernel scaffold; baseline (speedup 1.0000x reference)
import jax
import jax.numpy as jnp
from jax.experimental import pallas as pl


def kernel(node_feature, edge_index, edge_feature, nn_W, nn_b, root, bias, gamma, beta):
    raise NotImplementedError("write your pallas kernel here")


# speedup vs baseline: 1.3592x; 1.3592x over previous
import functools

import jax
import jax.numpy as jnp
from jax import lax
from jax.experimental import pallas as pl
from jax.experimental.pallas import tpu as pltpu
from jax.experimental.pallas import tpu_sc as plsc

_N = 10000
_E = 160000
_DI = 32
_DO = 32
_DE = 16

_NC = 2
_NS = 16
_NW = _NC * _NS
_CH = 128
_NCH = 40
_EPW = _CH * _NCH
_EPAD = _EPW * _NW
_ACC_N = 10016
_RPT = _ACC_N // _NS

_mesh = plsc.VectorSubcoreMesh(
    core_axis_name="c", subcore_axis_name="s", num_cores=_NC, num_subcores=_NS
)


@functools.partial(
    pl.kernel,
    out_type=jax.ShapeDtypeStruct((_EPAD, _DI), jnp.float32),
    mesh=_mesh,
    scratch_types=[
        pltpu.VMEM((_NCH, _CH), jnp.int32),
        pltpu.VMEM((_CH, _DI), jnp.float32),
        pltpu.SemaphoreType.DMA,
    ],
    compiler_params=pltpu.CompilerParams(use_tc_tiling_on_sc=False),
)
def _sc_gather(x_hbm, src_hbm, out_hbm, idx_v, rows_v, sem):
    cid = lax.axis_index("c")
    sid = lax.axis_index("s")
    wid = sid * _NC + cid
    pltpu.sync_copy(src_hbm.at[wid], idx_v)

    def body(c, _):
        pltpu.async_copy(x_hbm.at[idx_v.at[c]], rows_v, sem).wait()
        pltpu.sync_copy(rows_v, out_hbm.at[pl.ds(wid * _EPW + c * _CH, _CH)])
        return 0

    lax.fori_loop(0, _NCH, body, 0)


@functools.partial(
    pl.kernel,
    out_type=jax.ShapeDtypeStruct((_NC, _ACC_N, _DO), jnp.float32),
    mesh=_mesh,
    scratch_types=[
        pltpu.VMEM((_CH,), jnp.int32),
        pltpu.VMEM((_CH, _DO), jnp.float32),
        pltpu.VMEM_SHARED((_ACC_N, _DO), jnp.float32),
    ],
    compiler_params=pltpu.CompilerParams(use_tc_tiling_on_sc=False),
)
def _sc_scatter(msg_hbm, dst_hbm, zeros_hbm, out_hbm, idx_v, msg_v, acc_sh):
    cid = lax.axis_index("c")
    sid = lax.axis_index("s")
    wid = sid * _NC + cid

    @pl.when(sid == 0)
    def _():
        pltpu.sync_copy(zeros_hbm, acc_sh)

    plsc.subcore_barrier()

    def body(c, _):
        pltpu.sync_copy(dst_hbm.at[wid, c], idx_v)
        pltpu.sync_copy(msg_hbm.at[pl.ds(wid * _EPW + c * _CH, _CH)], msg_v)
        pltpu.sync_copy(msg_v, acc_sh.at[idx_v], add=True)
        return 0

    lax.fori_loop(0, _NCH, body, 0)
    plsc.subcore_barrier()

    @pl.when(sid == 0)
    def _():
        pltpu.sync_copy(acc_sh, out_hbm.at[cid])


_ET = 2048


def _msg_body(xj_ref, ef_ref, w_ref, out_ref):
    xj = xj_ref[...]
    ef = ef_ref[...]
    u = jnp.concatenate([ef[:, k : k + 1] * xj for k in range(_DE)] + [xj], axis=1)
    out_ref[...] = jnp.dot(
        u, w_ref[...], preferred_element_type=jnp.float32,
        precision=lax.Precision.HIGHEST,
    )


def _tc_msg(xj, ef_pad, wstack):
    return pl.pallas_call(
        _msg_body,
        grid=(_EPAD // _ET,),
        in_specs=[
            pl.BlockSpec((_ET, _DI), lambda i: (i, 0)),
            pl.BlockSpec((_ET, _DE), lambda i: (i, 0)),
            pl.BlockSpec(((_DE + 1) * _DI, _DO), lambda i: (0, 0)),
        ],
        out_specs=pl.BlockSpec((_ET, _DO), lambda i: (i, 0)),
        out_shape=jax.ShapeDtypeStruct((_EPAD, _DO), jnp.float32),
    )(xj, ef_pad, wstack)


def _final_body(a_ref, x_ref, root_ref, bias_ref, gamma_ref, beta_ref, out_ref):
    h = (
        a_ref[0]
        + a_ref[1]
        + jnp.dot(
            x_ref[...], root_ref[...], preferred_element_type=jnp.float32,
            precision=lax.Precision.HIGHEST,
        )
        + bias_ref[...]
    )
    mean = jnp.mean(h, axis=0, keepdims=True)
    hc = h - mean
    var = jnp.mean(hc * hc, axis=0, keepdims=True)
    hn = hc * lax.rsqrt(var + 1e-5) * gamma_ref[...] + beta_ref[...]
    out_ref[...] = jnp.where(hn >= 0.0, hn, 0.01 * hn)


def _tc_final(aggr, x, root, bias, gamma, beta):
    return pl.pallas_call(
        _final_body,
        out_shape=jax.ShapeDtypeStruct((_N, _DO), jnp.float32),
    )(aggr, x, root, bias, gamma, beta)


def kernel(node_feature, edge_index, edge_feature, nn_W, nn_b, root, bias, gamma, beta):
    pad = _EPAD - _E
    src = jnp.pad(edge_index[0], (0, pad)).reshape(_NW, _NCH, _CH)
    dst = jnp.pad(edge_index[1], (0, pad), constant_values=_N).reshape(_NW, _NCH, _CH)
    ef_pad = jnp.pad(edge_feature, ((0, pad), (0, 0)))
    wstack = jnp.concatenate(
        [nn_W.reshape(_DE * _DI, _DO), nn_b.reshape(_DI, _DO)], axis=0
    )
    zeros = jnp.zeros((_ACC_N, _DO), jnp.float32)

    xj = _sc_gather(node_feature, src)
    msg = _tc_msg(xj, ef_pad, wstack)
    partials = _sc_scatter(msg, dst, zeros)
    aggr = partials[:, :_N, :]
    return _tc_final(
        aggr,
        node_feature,
        root,
        bias.reshape(1, _DO),
        gamma.reshape(1, _DO),
        beta.reshape(1, _DO),
    )


# speedup vs baseline: 3.5835x; 2.6364x over previous
import functools

import jax
import jax.numpy as jnp
from jax import lax
from jax.experimental import pallas as pl
from jax.experimental.pallas import tpu as pltpu
from jax.experimental.pallas import tpu_sc as plsc

_N = 10000
_E = 160000
_DI = 32
_DO = 32
_DE = 16

_NC = 2
_NS = 16
_NW = _NC * _NS
_CH = 128
_NCH = 40
_EPW = _CH * _NCH
_EPAD = _EPW * _NW
_ACC_N = 10016
_RPT = _ACC_N // _NS

_mesh = plsc.VectorSubcoreMesh(
    core_axis_name="c", subcore_axis_name="s", num_cores=_NC, num_subcores=_NS
)


@functools.partial(
    pl.kernel,
    out_type=jax.ShapeDtypeStruct((_EPAD, _DI), jnp.float32),
    mesh=_mesh,
    scratch_types=[
        pltpu.VMEM((_NCH, _CH), jnp.int32),
        pltpu.VMEM((_CH, _DI), jnp.float32),
        pltpu.SemaphoreType.DMA,
    ],
    compiler_params=pltpu.CompilerParams(use_tc_tiling_on_sc=False),
)
def _sc_gather(x_hbm, src_hbm, out_hbm, idx_v, rows_v, sem):
    cid = lax.axis_index("c")
    sid = lax.axis_index("s")
    wid = sid * _NC + cid
    pltpu.sync_copy(src_hbm.at[wid], idx_v)

    def body(c, _):
        pltpu.async_copy(x_hbm.at[idx_v.at[c]], rows_v, sem).wait()
        pltpu.sync_copy(rows_v, out_hbm.at[pl.ds(wid * _EPW + c * _CH, _CH)])
        return 0

    lax.fori_loop(0, _NCH, body, 0)


@functools.partial(
    pl.kernel,
    out_type=jax.ShapeDtypeStruct((_NC, _ACC_N, _DO), jnp.float32),
    mesh=_mesh,
    scratch_types=[
        pltpu.VMEM((_CH,), jnp.int32),
        pltpu.VMEM((_CH, _DO), jnp.float32),
        pltpu.VMEM_SHARED((_ACC_N, _DO), jnp.float32),
    ],
    compiler_params=pltpu.CompilerParams(use_tc_tiling_on_sc=False),
)
def _sc_scatter(msg_hbm, dst_hbm, zeros_hbm, out_hbm, idx_v, msg_v, acc_sh):
    cid = lax.axis_index("c")
    sid = lax.axis_index("s")
    wid = sid * _NC + cid

    @pl.when(sid == 0)
    def _():
        pltpu.sync_copy(zeros_hbm, acc_sh)

    plsc.subcore_barrier()

    def body(c, _):
        pltpu.sync_copy(dst_hbm.at[wid, c], idx_v)
        pltpu.sync_copy(msg_hbm.at[pl.ds(wid * _EPW + c * _CH, _CH)], msg_v)
        pltpu.sync_copy(msg_v, acc_sh.at[idx_v], add=True)
        return 0

    lax.fori_loop(0, _NCH, body, 0)
    plsc.subcore_barrier()

    @pl.when(sid == 0)
    def _():
        pltpu.sync_copy(acc_sh, out_hbm.at[cid])


_ET = 4096


def _msg_body(xj_ref, ef_ref, wT_ref, out_ref):
    xjT = xj_ref[...].T
    efT = ef_ref[...].T
    uT = jnp.concatenate(
        [efT[k : k + 1, :] * xjT for k in range(_DE)] + [xjT], axis=0
    )
    msgT = jnp.dot(wT_ref[...], uT, preferred_element_type=jnp.float32)
    out_ref[...] = msgT.T


def _tc_msg(xj, ef_pad, wstack_T):
    return pl.pallas_call(
        _msg_body,
        grid=(_EPAD // _ET,),
        in_specs=[
            pl.BlockSpec((_ET, _DI), lambda i: (i, 0)),
            pl.BlockSpec((_ET, _DE), lambda i: (i, 0)),
            pl.BlockSpec((_DO, (_DE + 1) * _DI), lambda i: (0, 0)),
        ],
        out_specs=pl.BlockSpec((_ET, _DO), lambda i: (i, 0)),
        out_shape=jax.ShapeDtypeStruct((_EPAD, _DO), jnp.float32),
    )(xj, ef_pad, wstack_T)


def _final_body(a_ref, x_ref, root_ref, bias_ref, gamma_ref, beta_ref, out_ref):
    h = (
        a_ref[0]
        + a_ref[1]
        + jnp.dot(
            x_ref[...], root_ref[...], preferred_element_type=jnp.float32,
            precision=lax.Precision.HIGHEST,
        )
        + bias_ref[...]
    )
    mean = jnp.mean(h, axis=0, keepdims=True)
    hc = h - mean
    var = jnp.mean(hc * hc, axis=0, keepdims=True)
    hn = hc * lax.rsqrt(var + 1e-5) * gamma_ref[...] + beta_ref[...]
    out_ref[...] = jnp.where(hn >= 0.0, hn, 0.01 * hn)


def _tc_final(aggr, x, root, bias, gamma, beta):
    return pl.pallas_call(
        _final_body,
        out_shape=jax.ShapeDtypeStruct((_N, _DO), jnp.float32),
    )(aggr, x, root, bias, gamma, beta)


def kernel(node_feature, edge_index, edge_feature, nn_W, nn_b, root, bias, gamma, beta):
    pad = _EPAD - _E
    src = jnp.pad(edge_index[0], (0, pad)).reshape(_NW, _NCH, _CH)
    dst = jnp.pad(edge_index[1], (0, pad), constant_values=_N).reshape(_NW, _NCH, _CH)
    ef_pad = jnp.pad(edge_feature, ((0, pad), (0, 0)))
    wstack_T = jnp.concatenate(
        [nn_W.reshape(_DE * _DI, _DO), nn_b.reshape(_DI, _DO)], axis=0
    ).T
    zeros = jnp.zeros((_ACC_N, _DO), jnp.float32)

    xj = _sc_gather(node_feature, src)
    msg = _tc_msg(xj, ef_pad, wstack_T)
    partials = _sc_scatter(msg, dst, zeros)
    aggr = partials[:, :_N, :]
    return _tc_final(
        aggr,
        node_feature,
        root,
        bias.reshape(1, _DO),
        gamma.reshape(1, _DO),
        beta.reshape(1, _DO),
    )
